# baseline (speedup 1.0000x reference)
import functools

import jax
import jax.numpy as jnp
from jax import lax
from jax.experimental import pallas as pl
from jax.experimental.pallas import tpu as pltpu
from jax.experimental.pallas import tpu_sc as plsc

N = 10000
NPAD = 10240
L = 320000
NW = 32
BB = 128
NBL = 79
LW = NBL * BB
LPAD = NW * LW
NB = 2 * NBL
EPAD = NW * NB * BB
T = NPAD * NPAD
RS = NPAD // 16
G = 8
NGL, NTL = NBL // G, NBL % G
GW = 16
NGW, NTW = NBL // GW, NBL % GW
GA = 2
NGA = NB // GA

_mesh = plsc.VectorSubcoreMesh(core_axis_name="c", subcore_axis_name="s")


@functools.partial(
    pl.kernel,
    out_type=jax.ShapeDtypeStruct((T,), jnp.int32),
    mesh=_mesh,
    scratch_types=[
        pltpu.VMEM((NBL, BB), jnp.int32),
        pltpu.VMEM((NBL, BB), jnp.int32),
        pltpu.SemaphoreType.DMA,
    ],
)
def _sc_write_ids(keys_hbm, ids_hbm, table_hbm, keys_v, ids_v, sem):
    w = lax.axis_index("s") * 2 + lax.axis_index("c")
    pltpu.sync_copy(keys_hbm.at[w], keys_v)
    pltpu.sync_copy(ids_hbm.at[w], ids_v)

    def fire(j):
        return pltpu.async_copy(ids_v.at[j], table_hbm.at[keys_v.at[j]], sem)

    def group(g, _):
        descs = [fire(g * GW + b) for b in range(GW)]
        for d in descs:
            d.wait()
        return 0

    lax.fori_loop(0, NGW, group, 0)
    descs = [fire(NGW * GW + b) for b in range(NTW)]
    for d in descs:
        d.wait()


@functools.partial(
    pl.kernel,
    out_type=(
        jax.ShapeDtypeStruct((NW, NB, BB), jnp.int32),
        jax.ShapeDtypeStruct((2, NPAD), jnp.float32),
    ),
    mesh=_mesh,
    scratch_types=[
        pltpu.VMEM((NBL, BB), jnp.int32),
        pltpu.VMEM((NBL, BB), jnp.int32),
        pltpu.VMEM((NB, BB), jnp.int32),
        pltpu.VMEM((NB, BB), jnp.int32),
        pltpu.VMEM((NB, BB), jnp.float32),
        pltpu.VMEM_SHARED((NPAD,), jnp.float32),
        pltpu.SemaphoreType.DMA,
    ],
)
def _sc_readback_deg(keys_hbm, srce_hbm, dste_hbm, table_hbm, zeros_hbm,
                     src2_hbm, degp_hbm,
                     keys_v, rb_v, src_v, dst_v, vals_v, acc_sh, sem):
    c = lax.axis_index("c")
    s = lax.axis_index("s")
    w = s * 2 + c
    pltpu.sync_copy(keys_hbm.at[w], keys_v)
    pltpu.sync_copy(srce_hbm.at[w], src_v)
    pltpu.sync_copy(dste_hbm.at[w], dst_v)
    pltpu.sync_copy(zeros_hbm.at[pl.ds(s * RS, RS)], acc_sh.at[pl.ds(s * RS, RS)])

    def fire_rb(j):
        return pltpu.async_copy(table_hbm.at[keys_v.at[j]], rb_v.at[j], sem)

    def rb_group(g, _):
        descs = [fire_rb(g * G + b) for b in range(G)]
        for d in descs:
            d.wait()
        return 0

    lax.fori_loop(0, NGL, rb_group, 0)
    descs = [fire_rb(NGL * G + b) for b in range(NTL)]
    for d in descs:
        d.wait()

    iota = lax.iota(jnp.int32, 16)
    pad_rows = N + iota
    one = jnp.ones((16,), jnp.float32)
    zero = jnp.zeros((16,), jnp.float32)

    def body(j, _):
        base = w * LW + j * BB
        for k in range(BB // 16):
            sl = pl.ds(k * 16, 16)
            lid = base + k * 16 + iota
            a = src_v[j, sl]
            b = dst_v[j, sl]
            keep_f = rb_v[j, sl] == lid
            keep_r = keep_f & (a != b)
            src_v[j, sl] = jnp.where(keep_f, a, pad_rows)
            src_v[NBL + j, sl] = jnp.where(keep_r, b, pad_rows)
            vals_v[j, sl] = jnp.where(keep_f, one, zero)
            vals_v[NBL + j, sl] = jnp.where(keep_r, one, zero)
        return 0

    lax.fori_loop(0, NBL, body, 0)
    pltpu.sync_copy(src_v, src2_hbm.at[w])
    plsc.subcore_barrier()

    def fire_deg(j):
        return pltpu.async_copy(vals_v.at[j], acc_sh.at[dst_v.at[j]], sem,
                                add=True)

    def deg_group(g, _):
        descs = [fire_deg(g * G + b) for b in range(G)]
        for d in descs:
            d.wait()
        return 0

    lax.fori_loop(0, NB // G, deg_group, 0)
    descs = [fire_deg((NB // G) * G + b) for b in range(NB % G)]
    for d in descs:
        d.wait()

    plsc.subcore_barrier()
    pltpu.sync_copy(acc_sh.at[pl.ds(s * RS, RS)], degp_hbm.at[c].at[pl.ds(s * RS, RS)])


ACCROWS = 2688
DUMROW = 2560
RSA = ACCROWS // 16


def _make_sc_agg(P, ROUNDS):
    SHIFT = {2: 1, 4: 2}[P]
    LOCR = NPAD // ROUNDS

    @functools.partial(
        pl.kernel,
        out_type=jax.ShapeDtypeStruct((ROUNDS, 2, ACCROWS, 128), jnp.float32),
        mesh=_mesh,
        scratch_types=[
            pltpu.VMEM((NB, BB), jnp.int32),
            pltpu.VMEM((NB, BB), jnp.int32),
            pltpu.VMEM((2, GA, BB, 128), jnp.float32),
            pltpu.VMEM_SHARED((ACCROWS, 128), jnp.float32),
            pltpu.SemaphoreType.DMA,
            pltpu.SemaphoreType.DMA,
        ],
    )
    def agg(gtbl_hbm, src2_hbm, dst_hbm, zeros_hbm, accp_hbm,
            tix_v, row_v, rows_v, acc_sh, sem, ssem):
        c = lax.axis_index("c")
        s = lax.axis_index("s")
        w = s * 2 + c
        iota = lax.iota(jnp.int32, 16)
        dummy = DUMROW + iota

        for r in range(ROUNDS):
            pltpu.sync_copy(zeros_hbm.at[pl.ds(s * RSA, RSA)],
                            acc_sh.at[pl.ds(s * RSA, RSA)])
            pltpu.sync_copy(src2_hbm.at[w], tix_v)
            pltpu.sync_copy(dst_hbm.at[w], row_v)
            plsc.subcore_barrier()

            def compute(j, _):
                for k in range(BB // 16):
                    sl = pl.ds(k * 16, 16)
                    local = row_v[j, sl] - r * LOCR
                    inr = (local >= 0) & (local < LOCR)
                    row_v[j, sl] = jnp.where(inr, local >> SHIFT, dummy)
                    tix_v[j, sl] = tix_v[j, sl] + jnp.where(
                        inr, (local & (P - 1)) * NPAD, 0)
                return 0

            lax.fori_loop(0, NB, compute, 0)

            def fire(g):
                return [
                    pltpu.async_copy(gtbl_hbm.at[tix_v.at[g * GA + b]],
                                     rows_v.at[g % 2].at[b], sem)
                    for b in range(GA)
                ]

            def scatter(g, descs):
                sds = []
                for b in range(GA):
                    descs[b].wait()
                    sds.append(pltpu.async_copy(
                        rows_v.at[g % 2].at[b],
                        acc_sh.at[row_v.at[g * GA + b]], ssem, add=True))
                return sds

            prev = fire(0)
            sprev = None
            for g in range(1, NGA):
                if sprev is not None:
                    for d in sprev:
                        d.wait()
                cur = fire(g)
                sprev = scatter(g - 1, prev)
                prev = cur
            for d in sprev:
                d.wait()
            for d in scatter(NGA - 1, prev):
                d.wait()

            plsc.subcore_barrier()
            pltpu.sync_copy(acc_sh.at[pl.ds(s * RSA, RSA)],
                            accp_hbm.at[r].at[c].at[pl.ds(s * RSA, RSA)])
            plsc.subcore_barrier()

    return agg


_sc_agg1 = _make_sc_agg(2, 2)
_sc_agg2 = _make_sc_agg(4, 1)


def _dinv_from(degp):
    deg = degp[0] + degp[1] + 1.0
    return lax.rsqrt(deg)


def _tc_pre(xpad_ref, w1_ref, degp_ref, g1_ref):
    dinv = _dinv_from(degp_ref[...])
    h = jnp.dot(xpad_ref[...], w1_ref[...], preferred_element_type=jnp.float32)
    g = h * dinv[:, None]
    z = jnp.zeros((NPAD, 64), jnp.float32)
    g1_ref[...] = jnp.concatenate(
        [jnp.concatenate([g, z], axis=1),
         jnp.concatenate([z, g], axis=1)], axis=0)


def _bn_relu(a, gamma, beta):
    mu = jnp.mean(a, axis=0)
    var = jnp.mean((a - mu) ** 2, axis=0)
    return jnp.maximum((a - mu) / jnp.sqrt(var + 1e-5) * gamma + beta, 0.0)


def _tc_mid(acc_ref, g1_ref, degp_ref, b1_ref, gm1_ref, bt1_ref, w2_ref,
            h1f_ref, g2_ref):
    dinv = _dinv_from(degp_ref[...])
    ssum = acc_ref[...] + g1_ref[:NPAD, :64]
    agg = ssum * dinv[:, None] + b1_ref[...]
    h1f = _bn_relu(agg[:N], gm1_ref[...], bt1_ref[...])
    h1f_ref[...] = h1f
    h2 = jnp.dot(h1f, w2_ref[...], preferred_element_type=jnp.float32)
    g = jnp.concatenate(
        [h2 * dinv[:N, None], jnp.zeros((NPAD - N, 32), jnp.float32)], axis=0)
    z = jnp.zeros((NPAD, 32), jnp.float32)
    g2_ref[...] = jnp.concatenate(
        [jnp.concatenate([g, z, z, z], axis=1),
         jnp.concatenate([z, g, z, z], axis=1),
         jnp.concatenate([z, z, g, z], axis=1),
         jnp.concatenate([z, z, z, g], axis=1)], axis=0)


def _tc_head(acc_ref, g2_ref, degp_ref, b2_ref, gm2_ref, bt2_ref,
             x_ref, h1f_ref, wf_ref, bf_ref, wj1_ref, bj1_ref, wj2_ref,
             bj2_ref, out_ref):
    dinv = _dinv_from(degp_ref[...])
    ssum = acc_ref[...] + g2_ref[:NPAD, :32]
    agg = ssum * dinv[:, None] + b2_ref[...]
    h2f = _bn_relu(agg[:N], gm2_ref[...], bt2_ref[...])
    cat = jnp.concatenate([x_ref[...], h1f_ref[...], h2f], axis=1)
    jf = jnp.maximum(
        jnp.dot(cat, wf_ref[...], preferred_element_type=jnp.float32)
        + bf_ref[...], 0.0)
    t = jnp.maximum(
        jnp.dot(jf, wj1_ref[...], preferred_element_type=jnp.float32)
        + bj1_ref[...], 0.0)
    out_ref[...] = (jnp.dot(t, wj2_ref[...], preferred_element_type=jnp.float32)
                    + bj2_ref[...])


def kernel(line_features, junction_features, line2junction_idx,
           junction_logits, line_logits,
           W1, b1, g1, be1, W2, b2, g2, be2, Wf, bf, Wj1, bj1, Wj2, bj2):
    f32 = jnp.float32
    i32 = jnp.int32

    pad_idx = (N + (jnp.arange(LPAD - L, dtype=i32) % 16)).astype(i32)
    A = jnp.concatenate([line2junction_idx[:, 0], pad_idx]).reshape(NW, NBL, BB)
    B = jnp.concatenate([line2junction_idx[:, 1], pad_idx]).reshape(NW, NBL, BB)
    keys = jnp.minimum(A, B) * NPAD + jnp.maximum(A, B)
    ids = jnp.arange(LPAD, dtype=i32).reshape(NW, NBL, BB)
    srce = jnp.concatenate([A, B], axis=1)
    dste = jnp.concatenate([B, A], axis=1)

    xpad = jnp.concatenate([junction_features, jnp.zeros((NPAD - N, 128), f32)])
    zeros1 = jnp.zeros((NPAD,), f32)
    zacc = jnp.zeros((ACCROWS, 128), f32)
    wj2p = jnp.concatenate([Wj2, jnp.zeros((32, 128 - 3), f32)], axis=1)
    bj2p = jnp.concatenate([bj2, jnp.zeros((128 - 3,), f32)])

    table = _sc_write_ids(keys, ids)
    src2, degp = _sc_readback_deg(keys, srce, dste, table, zeros1)

    g1pad = pl.pallas_call(
        _tc_pre,
        out_shape=jax.ShapeDtypeStruct((2 * NPAD, 128), f32),
    )(xpad, W1, degp)

    acc1p = _sc_agg1(g1pad, src2, dste, zacc)
    acc1 = (acc1p[:, 0] + acc1p[:, 1]).reshape(2, ACCROWS * 2, 64)
    acc1 = acc1[:, : NPAD // 2].reshape(NPAD, 64)

    h1f, g2pad = pl.pallas_call(
        _tc_mid,
        out_shape=(
            jax.ShapeDtypeStruct((N, 64), f32),
            jax.ShapeDtypeStruct((4 * NPAD, 128), f32),
        ),
    )(acc1, g1pad, degp, b1.reshape(1, 64), g1.reshape(1, 64),
      be1.reshape(1, 64), W2)

    acc2p = _sc_agg2(g2pad, src2, dste, zacc)
    acc2 = (acc2p[0, 0] + acc2p[0, 1]).reshape(ACCROWS * 4, 32)[:NPAD]

    out = pl.pallas_call(
        _tc_head,
        out_shape=jax.ShapeDtypeStruct((N, 128), f32),
    )(acc2, g2pad, degp, b2.reshape(1, 32), g2.reshape(1, 32),
      be2.reshape(1, 32), junction_features, h1f, Wf, bf.reshape(1, 128),
      Wj1, bj1.reshape(1, 32), wj2p, bj2p.reshape(1, 128))

    return (line_logits, out[:, :3])


# speedup vs baseline: 1.0088x; 1.0088x over previous
import functools

import jax
import jax.numpy as jnp
from jax import lax
from jax.experimental import pallas as pl
from jax.experimental.pallas import tpu as pltpu
from jax.experimental.pallas import tpu_sc as plsc

N = 10000
NPAD = 10240
L = 320000
NW = 32
BB = 128
NBL = 79
LW = NBL * BB
LPAD = NW * LW
NB = 2 * NBL
EPAD = NW * NB * BB
T = NPAD * NPAD
RS = NPAD // 16
G = 8
NGL, NTL = NBL // G, NBL % G
GW = 16
NGW, NTW = NBL // GW, NBL % GW
GA = 2
NGA = NB // GA

_mesh = plsc.VectorSubcoreMesh(core_axis_name="c", subcore_axis_name="s")


@functools.partial(
    pl.kernel,
    out_type=jax.ShapeDtypeStruct((T,), jnp.int32),
    mesh=_mesh,
    scratch_types=[
        pltpu.VMEM((NBL, BB), jnp.int32),
        pltpu.VMEM((NBL, BB), jnp.int32),
        pltpu.SemaphoreType.DMA,
    ],
)
def _sc_write_ids(keys_hbm, ids_hbm, table_hbm, keys_v, ids_v, sem):
    w = lax.axis_index("s") * 2 + lax.axis_index("c")
    pltpu.sync_copy(keys_hbm.at[w], keys_v)
    pltpu.sync_copy(ids_hbm.at[w], ids_v)

    def fire(j):
        return pltpu.async_copy(ids_v.at[j], table_hbm.at[keys_v.at[j]], sem)

    def group(g, _):
        descs = [fire(g * G + b) for b in range(G)]
        for d in descs:
            d.wait()
        return 0

    lax.fori_loop(0, NGL, group, 0)
    descs = [fire(NGL * G + b) for b in range(NTL)]
    for d in descs:
        d.wait()


@functools.partial(
    pl.kernel,
    out_type=(
        jax.ShapeDtypeStruct((NW, NB, BB), jnp.int32),
        jax.ShapeDtypeStruct((2, NPAD), jnp.float32),
    ),
    mesh=_mesh,
    scratch_types=[
        pltpu.VMEM((NBL, BB), jnp.int32),
        pltpu.VMEM((NBL, BB), jnp.int32),
        pltpu.VMEM((NB, BB), jnp.int32),
        pltpu.VMEM((NB, BB), jnp.int32),
        pltpu.VMEM((NB, BB), jnp.float32),
        pltpu.VMEM_SHARED((NPAD,), jnp.float32),
        pltpu.SemaphoreType.DMA,
    ],
)
def _sc_readback_deg(keys_hbm, srce_hbm, dste_hbm, table_hbm, zeros_hbm,
                     src2_hbm, degp_hbm,
                     keys_v, rb_v, src_v, dst_v, vals_v, acc_sh, sem):
    c = lax.axis_index("c")
    s = lax.axis_index("s")
    w = s * 2 + c
    pltpu.sync_copy(keys_hbm.at[w], keys_v)
    pltpu.sync_copy(srce_hbm.at[w], src_v)
    pltpu.sync_copy(dste_hbm.at[w], dst_v)
    pltpu.sync_copy(zeros_hbm.at[pl.ds(s * RS, RS)], acc_sh.at[pl.ds(s * RS, RS)])

    def fire_rb(j):
        return pltpu.async_copy(table_hbm.at[keys_v.at[j]], rb_v.at[j], sem)

    def rb_group(g, _):
        descs = [fire_rb(g * G + b) for b in range(G)]
        for d in descs:
            d.wait()
        return 0

    lax.fori_loop(0, NGL, rb_group, 0)
    descs = [fire_rb(NGL * G + b) for b in range(NTL)]
    for d in descs:
        d.wait()

    iota = lax.iota(jnp.int32, 16)
    pad_rows = N + iota
    one = jnp.ones((16,), jnp.float32)
    zero = jnp.zeros((16,), jnp.float32)

    def body(j, _):
        base = w * LW + j * BB
        for k in range(BB // 16):
            sl = pl.ds(k * 16, 16)
            lid = base + k * 16 + iota
            a = src_v[j, sl]
            b = dst_v[j, sl]
            keep_f = rb_v[j, sl] == lid
            keep_r = keep_f & (a != b)
            src_v[j, sl] = jnp.where(keep_f, a, pad_rows)
            src_v[NBL + j, sl] = jnp.where(keep_r, b, pad_rows)
            vals_v[j, sl] = jnp.where(keep_f, one, zero)
            vals_v[NBL + j, sl] = jnp.where(keep_r, one, zero)
        return 0

    lax.fori_loop(0, NBL, body, 0)
    pltpu.sync_copy(src_v, src2_hbm.at[w])
    plsc.subcore_barrier()

    def fire_deg(j):
        return pltpu.async_copy(vals_v.at[j], acc_sh.at[dst_v.at[j]], sem,
                                add=True)

    def deg_group(g, _):
        descs = [fire_deg(g * G + b) for b in range(G)]
        for d in descs:
            d.wait()
        return 0

    lax.fori_loop(0, NB // G, deg_group, 0)
    descs = [fire_deg((NB // G) * G + b) for b in range(NB % G)]
    for d in descs:
        d.wait()

    plsc.subcore_barrier()
    pltpu.sync_copy(acc_sh.at[pl.ds(s * RS, RS)], degp_hbm.at[c].at[pl.ds(s * RS, RS)])


ACCROWS = 2688
DUMROW = 2560
RSA = ACCROWS // 16


def _make_sc_agg(P, ROUNDS):
    SHIFT = {2: 1, 4: 2}[P]
    LOCR = NPAD // ROUNDS

    @functools.partial(
        pl.kernel,
        out_type=jax.ShapeDtypeStruct((ROUNDS, 2, ACCROWS, 128), jnp.float32),
        mesh=_mesh,
        scratch_types=[
            pltpu.VMEM((NB, BB), jnp.int32),
            pltpu.VMEM((NB, BB), jnp.int32),
            pltpu.VMEM((2, GA, BB, 128), jnp.float32),
            pltpu.VMEM_SHARED((ACCROWS, 128), jnp.float32),
            pltpu.SemaphoreType.DMA,
        ],
    )
    def agg(gtbl_hbm, src2_hbm, dst_hbm, zeros_hbm, accp_hbm,
            tix_v, row_v, rows_v, acc_sh, sem):
        c = lax.axis_index("c")
        s = lax.axis_index("s")
        w = s * 2 + c
        iota = lax.iota(jnp.int32, 16)
        dummy = DUMROW + iota

        for r in range(ROUNDS):
            pltpu.sync_copy(zeros_hbm.at[pl.ds(s * RSA, RSA)],
                            acc_sh.at[pl.ds(s * RSA, RSA)])
            pltpu.sync_copy(src2_hbm.at[w], tix_v)
            pltpu.sync_copy(dst_hbm.at[w], row_v)
            plsc.subcore_barrier()

            def compute(j, _):
                for k in range(BB // 16):
                    sl = pl.ds(k * 16, 16)
                    local = row_v[j, sl] - r * LOCR
                    inr = (local >= 0) & (local < LOCR)
                    row_v[j, sl] = jnp.where(inr, local >> SHIFT, dummy)
                    tix_v[j, sl] = tix_v[j, sl] + jnp.where(
                        inr, (local & (P - 1)) * NPAD, 0)
                return 0

            lax.fori_loop(0, NB, compute, 0)

            def fire(g):
                return [
                    pltpu.async_copy(gtbl_hbm.at[tix_v.at[g * GA + b]],
                                     rows_v.at[g % 2].at[b], sem)
                    for b in range(GA)
                ]

            def drain_scatter(g, descs):
                for b in range(GA):
                    descs[b].wait()
                    pltpu.sync_copy(rows_v.at[g % 2].at[b],
                                    acc_sh.at[row_v.at[g * GA + b]], add=True)

            prev = fire(0)
            for g in range(1, NGA):
                cur = fire(g)
                drain_scatter(g - 1, prev)
                prev = cur
            drain_scatter(NGA - 1, prev)

            plsc.subcore_barrier()
            pltpu.sync_copy(acc_sh.at[pl.ds(s * RSA, RSA)],
                            accp_hbm.at[r].at[c].at[pl.ds(s * RSA, RSA)])
            plsc.subcore_barrier()

    return agg


_sc_agg1 = _make_sc_agg(2, 2)
_sc_agg2 = _make_sc_agg(4, 1)


def _dinv_from(degp):
    deg = degp[0] + degp[1] + 1.0
    return lax.rsqrt(deg)


def _tc_pre(xpad_ref, w1_ref, degp_ref, g1_ref):
    dinv = _dinv_from(degp_ref[...])
    h = jnp.dot(xpad_ref[...], w1_ref[...], preferred_element_type=jnp.float32)
    g = h * dinv[:, None]
    z = jnp.zeros((NPAD, 64), jnp.float32)
    g1_ref[...] = jnp.concatenate(
        [jnp.concatenate([g, z], axis=1),
         jnp.concatenate([z, g], axis=1)], axis=0)


def _bn_relu(a, gamma, beta):
    mu = jnp.mean(a, axis=0)
    var = jnp.mean((a - mu) ** 2, axis=0)
    return jnp.maximum((a - mu) / jnp.sqrt(var + 1e-5) * gamma + beta, 0.0)


def _tc_mid(acc_ref, g1_ref, degp_ref, b1_ref, gm1_ref, bt1_ref, w2_ref,
            h1f_ref, g2_ref):
    dinv = _dinv_from(degp_ref[...])
    ssum = acc_ref[...] + g1_ref[:NPAD, :64]
    agg = ssum * dinv[:, None] + b1_ref[...]
    h1f = _bn_relu(agg[:N], gm1_ref[...], bt1_ref[...])
    h1f_ref[...] = h1f
    h2 = jnp.dot(h1f, w2_ref[...], preferred_element_type=jnp.float32)
    g = jnp.concatenate(
        [h2 * dinv[:N, None], jnp.zeros((NPAD - N, 32), jnp.float32)], axis=0)
    z = jnp.zeros((NPAD, 32), jnp.float32)
    g2_ref[...] = jnp.concatenate(
        [jnp.concatenate([g, z, z, z], axis=1),
         jnp.concatenate([z, g, z, z], axis=1),
         jnp.concatenate([z, z, g, z], axis=1),
         jnp.concatenate([z, z, z, g], axis=1)], axis=0)


def _tc_head(acc_ref, g2_ref, degp_ref, b2_ref, gm2_ref, bt2_ref,
             x_ref, h1f_ref, wf_ref, bf_ref, wj1_ref, bj1_ref, wj2_ref,
             bj2_ref, out_ref):
    dinv = _dinv_from(degp_ref[...])
    ssum = acc_ref[...] + g2_ref[:NPAD, :32]
    agg = ssum * dinv[:, None] + b2_ref[...]
    h2f = _bn_relu(agg[:N], gm2_ref[...], bt2_ref[...])
    cat = jnp.concatenate([x_ref[...], h1f_ref[...], h2f], axis=1)
    jf = jnp.maximum(
        jnp.dot(cat, wf_ref[...], preferred_element_type=jnp.float32)
        + bf_ref[...], 0.0)
    t = jnp.maximum(
        jnp.dot(jf, wj1_ref[...], preferred_element_type=jnp.float32)
        + bj1_ref[...], 0.0)
    out_ref[...] = (jnp.dot(t, wj2_ref[...], preferred_element_type=jnp.float32)
                    + bj2_ref[...])


def kernel(line_features, junction_features, line2junction_idx,
           junction_logits, line_logits,
           W1, b1, g1, be1, W2, b2, g2, be2, Wf, bf, Wj1, bj1, Wj2, bj2):
    f32 = jnp.float32
    i32 = jnp.int32

    pad_idx = (N + (jnp.arange(LPAD - L, dtype=i32) % 16)).astype(i32)
    A = jnp.concatenate([line2junction_idx[:, 0], pad_idx]).reshape(NW, NBL, BB)
    B = jnp.concatenate([line2junction_idx[:, 1], pad_idx]).reshape(NW, NBL, BB)
    keys = jnp.minimum(A, B) * NPAD + jnp.maximum(A, B)
    ids = jnp.arange(LPAD, dtype=i32).reshape(NW, NBL, BB)
    srce = jnp.concatenate([A, B], axis=1)
    dste = jnp.concatenate([B, A], axis=1)

    xpad = jnp.concatenate([junction_features, jnp.zeros((NPAD - N, 128), f32)])
    zeros1 = jnp.zeros((NPAD,), f32)
    zacc = jnp.zeros((ACCROWS, 128), f32)
    wj2p = jnp.concatenate([Wj2, jnp.zeros((32, 128 - 3), f32)], axis=1)
    bj2p = jnp.concatenate([bj2, jnp.zeros((128 - 3,), f32)])

    table = _sc_write_ids(keys, ids)
    src2, degp = _sc_readback_deg(keys, srce, dste, table, zeros1)

    g1pad = pl.pallas_call(
        _tc_pre,
        out_shape=jax.ShapeDtypeStruct((2 * NPAD, 128), f32),
    )(xpad, W1, degp)

    acc1p = _sc_agg1(g1pad, src2, dste, zacc)
    acc1 = (acc1p[:, 0] + acc1p[:, 1]).reshape(2, ACCROWS * 2, 64)
    acc1 = acc1[:, : NPAD // 2].reshape(NPAD, 64)

    h1f, g2pad = pl.pallas_call(
        _tc_mid,
        out_shape=(
            jax.ShapeDtypeStruct((N, 64), f32),
            jax.ShapeDtypeStruct((4 * NPAD, 128), f32),
        ),
    )(acc1, g1pad, degp, b1.reshape(1, 64), g1.reshape(1, 64),
      be1.reshape(1, 64), W2)

    acc2p = _sc_agg2(g2pad, src2, dste, zacc)
    acc2 = (acc2p[0, 0] + acc2p[0, 1]).reshape(ACCROWS * 4, 32)[:NPAD]

    out = pl.pallas_call(
        _tc_head,
        out_shape=jax.ShapeDtypeStruct((N, 128), f32),
    )(acc2, g2pad, degp, b2.reshape(1, 32), g2.reshape(1, 32),
      be2.reshape(1, 32), junction_features, h1f, Wf, bf.reshape(1, 128),
      Wj1, bj1.reshape(1, 32), wj2p, bj2p.reshape(1, 128))

    return (line_logits, out[:, :3])


# speedup vs baseline: 1.0090x; 1.0002x over previous
import functools

import jax
import jax.numpy as jnp
from jax import lax
from jax.experimental import pallas as pl
from jax.experimental.pallas import tpu as pltpu
from jax.experimental.pallas import tpu_sc as plsc

N = 10000
NPAD = 10240
L = 320000
NW = 32
BB = 128
NBL = 79
LW = NBL * BB
LPAD = NW * LW
NB = 2 * NBL
EPAD = NW * NB * BB
T = NPAD * NPAD
RS = NPAD // 16
G = 8
NGL, NTL = NBL // G, NBL % G
GA = 2
NGA = NB // GA

_mesh = plsc.VectorSubcoreMesh(core_axis_name="c", subcore_axis_name="s")


@functools.partial(
    pl.kernel,
    out_type=jax.ShapeDtypeStruct((T,), jnp.int32),
    mesh=_mesh,
    scratch_types=[
        pltpu.VMEM((NBL, BB), jnp.int32),
        pltpu.VMEM((NBL, BB), jnp.int32),
        pltpu.SemaphoreType.DMA,
    ],
)
def _sc_write_ids(keys_hbm, ids_hbm, table_hbm, keys_v, ids_v, sem):
    w = lax.axis_index("s") * 2 + lax.axis_index("c")
    pltpu.sync_copy(keys_hbm.at[w], keys_v)
    pltpu.sync_copy(ids_hbm.at[w], ids_v)

    def fire(j):
        return pltpu.async_copy(ids_v.at[j], table_hbm.at[keys_v.at[j]], sem)

    def group(g, _):
        descs = [fire(g * G + b) for b in range(G)]
        for d in descs:
            d.wait()
        return 0

    lax.fori_loop(0, NGL, group, 0)
    descs = [fire(NGL * G + b) for b in range(NTL)]
    for d in descs:
        d.wait()


@functools.partial(
    pl.kernel,
    out_type=(
        jax.ShapeDtypeStruct((NW, NB, BB), jnp.int32),
        jax.ShapeDtypeStruct((2, NPAD), jnp.float32),
    ),
    mesh=_mesh,
    scratch_types=[
        pltpu.VMEM((NBL, BB), jnp.int32),
        pltpu.VMEM((NBL, BB), jnp.int32),
        pltpu.VMEM((NB, BB), jnp.int32),
        pltpu.VMEM((NB, BB), jnp.int32),
        pltpu.VMEM((NB, BB), jnp.float32),
        pltpu.VMEM_SHARED((NPAD,), jnp.float32),
        pltpu.SemaphoreType.DMA,
    ],
)
def _sc_readback_deg(keys_hbm, srce_hbm, dste_hbm, table_hbm, zeros_hbm,
                     src2_hbm, degp_hbm,
                     keys_v, rb_v, src_v, dst_v, vals_v, acc_sh, sem):
    c = lax.axis_index("c")
    s = lax.axis_index("s")
    w = s * 2 + c
    pltpu.sync_copy(keys_hbm.at[w], keys_v)
    pltpu.sync_copy(srce_hbm.at[w], src_v)
    pltpu.sync_copy(dste_hbm.at[w], dst_v)
    pltpu.sync_copy(zeros_hbm.at[pl.ds(s * RS, RS)], acc_sh.at[pl.ds(s * RS, RS)])

    def fire_rb(j):
        return pltpu.async_copy(table_hbm.at[keys_v.at[j]], rb_v.at[j], sem)

    def rb_group(g, _):
        descs = [fire_rb(g * G + b) for b in range(G)]
        for d in descs:
            d.wait()
        return 0

    lax.fori_loop(0, NGL, rb_group, 0)
    descs = [fire_rb(NGL * G + b) for b in range(NTL)]
    for d in descs:
        d.wait()

    iota = lax.iota(jnp.int32, 16)
    pad_rows = N + iota
    one = jnp.ones((16,), jnp.float32)
    zero = jnp.zeros((16,), jnp.float32)

    def body(j, _):
        base = w * LW + j * BB
        for k in range(BB // 16):
            sl = pl.ds(k * 16, 16)
            lid = base + k * 16 + iota
            a = src_v[j, sl]
            b = dst_v[j, sl]
            keep_f = rb_v[j, sl] == lid
            keep_r = keep_f & (a != b)
            src_v[j, sl] = jnp.where(keep_f, a, pad_rows)
            src_v[NBL + j, sl] = jnp.where(keep_r, b, pad_rows)
            vals_v[j, sl] = jnp.where(keep_f, one, zero)
            vals_v[NBL + j, sl] = jnp.where(keep_r, one, zero)
        return 0

    lax.fori_loop(0, NBL, body, 0)
    pltpu.sync_copy(src_v, src2_hbm.at[w])
    plsc.subcore_barrier()

    def fire_deg(j):
        return pltpu.async_copy(vals_v.at[j], acc_sh.at[dst_v.at[j]], sem,
                                add=True)

    def deg_group(g, _):
        descs = [fire_deg(g * G + b) for b in range(G)]
        for d in descs:
            d.wait()
        return 0

    lax.fori_loop(0, NB // G, deg_group, 0)
    descs = [fire_deg((NB // G) * G + b) for b in range(NB % G)]
    for d in descs:
        d.wait()

    plsc.subcore_barrier()
    pltpu.sync_copy(acc_sh.at[pl.ds(s * RS, RS)], degp_hbm.at[c].at[pl.ds(s * RS, RS)])


ACCROWS = 2688
DUMROW = 2560
RSA = ACCROWS // 16


def _make_sc_agg(P, ROUNDS):
    SHIFT = {2: 1, 4: 2}[P]
    LOCR = NPAD // ROUNDS

    @functools.partial(
        pl.kernel,
        out_type=jax.ShapeDtypeStruct((ROUNDS, 2, ACCROWS, 128), jnp.float32),
        mesh=_mesh,
        scratch_types=[
            pltpu.VMEM((NB, BB), jnp.int32),
            pltpu.VMEM((NB, BB), jnp.int32),
            pltpu.VMEM((2, GA, BB, 128), jnp.float32),
            pltpu.VMEM_SHARED((ACCROWS, 128), jnp.float32),
            pltpu.SemaphoreType.DMA,
        ],
    )
    def agg(gtbl_hbm, src2_hbm, dst_hbm, zeros_hbm, accp_hbm,
            tix_v, row_v, rows_v, acc_sh, sem):
        c = lax.axis_index("c")
        s = lax.axis_index("s")
        w = s * 2 + c
        iota = lax.iota(jnp.int32, 16)
        dummy = DUMROW + iota

        for r in range(ROUNDS):
            pltpu.sync_copy(zeros_hbm.at[pl.ds(s * RSA, RSA)],
                            acc_sh.at[pl.ds(s * RSA, RSA)])
            pltpu.sync_copy(src2_hbm.at[w], tix_v)
            pltpu.sync_copy(dst_hbm.at[w], row_v)
            plsc.subcore_barrier()

            def compute(j, _):
                for k in range(BB // 16):
                    sl = pl.ds(k * 16, 16)
                    local = row_v[j, sl] - r * LOCR
                    inr = (local >= 0) & (local < LOCR)
                    row_v[j, sl] = jnp.where(inr, local >> SHIFT, dummy)
                    tix_v[j, sl] = tix_v[j, sl] + jnp.where(
                        inr, (local & (P - 1)) * NPAD, 0)
                return 0

            lax.fori_loop(0, NB, compute, 0)

            def fire(g):
                return [
                    pltpu.async_copy(gtbl_hbm.at[tix_v.at[g * GA + b]],
                                     rows_v.at[g % 2].at[b], sem)
                    for b in range(GA)
                ]

            def drain_scatter(g, descs):
                for b in range(GA):
                    descs[b].wait()
                    pltpu.sync_copy(rows_v.at[g % 2].at[b],
                                    acc_sh.at[row_v.at[g * GA + b]], add=True)

            prev = fire(0)
            for g in range(1, NGA):
                cur = fire(g)
                drain_scatter(g - 1, prev)
                prev = cur
            drain_scatter(NGA - 1, prev)

            plsc.subcore_barrier()
            pltpu.sync_copy(acc_sh.at[pl.ds(s * RSA, RSA)],
                            accp_hbm.at[r].at[c].at[pl.ds(s * RSA, RSA)])
            plsc.subcore_barrier()

    return agg


_sc_agg1 = _make_sc_agg(2, 2)
_sc_agg2 = _make_sc_agg(4, 1)


def _dinv_from(degp):
    deg = degp[0] + degp[1] + 1.0
    return lax.rsqrt(deg)


def _tc_pre(xpad_ref, w1_ref, degp_ref, g1_ref):
    dinv = _dinv_from(degp_ref[...])
    h = jnp.dot(xpad_ref[...], w1_ref[...], preferred_element_type=jnp.float32)
    g = h * dinv[:, None]
    z = jnp.zeros((NPAD, 64), jnp.float32)
    g1_ref[...] = jnp.concatenate(
        [jnp.concatenate([g, z], axis=1),
         jnp.concatenate([z, g], axis=1)], axis=0)


def _bn_relu(a, gamma, beta):
    mu = jnp.mean(a, axis=0)
    var = jnp.mean((a - mu) ** 2, axis=0)
    return jnp.maximum((a - mu) / jnp.sqrt(var + 1e-5) * gamma + beta, 0.0)


def _tc_mid(acc_ref, g1_ref, degp_ref, b1_ref, gm1_ref, bt1_ref, w2_ref,
            h1f_ref, g2_ref):
    dinv = _dinv_from(degp_ref[...])
    ssum = acc_ref[...] + g1_ref[:NPAD, :64]
    agg = ssum * dinv[:, None] + b1_ref[...]
    h1f = _bn_relu(agg[:N], gm1_ref[...], bt1_ref[...])
    h1f_ref[...] = h1f
    h2 = jnp.dot(h1f, w2_ref[...], preferred_element_type=jnp.float32)
    g = jnp.concatenate(
        [h2 * dinv[:N, None], jnp.zeros((NPAD - N, 32), jnp.float32)], axis=0)
    z = jnp.zeros((NPAD, 32), jnp.float32)
    g2_ref[...] = jnp.concatenate(
        [jnp.concatenate([g, z, z, z], axis=1),
         jnp.concatenate([z, g, z, z], axis=1),
         jnp.concatenate([z, z, g, z], axis=1),
         jnp.concatenate([z, z, z, g], axis=1)], axis=0)


def _tc_head(acc_ref, g2_ref, degp_ref, b2_ref, gm2_ref, bt2_ref,
             x_ref, h1f_ref, wf_ref, bf_ref, wj1_ref, bj1_ref, wj2_ref,
             bj2_ref, out_ref):
    dinv = _dinv_from(degp_ref[...])
    ssum = acc_ref[...] + g2_ref[:NPAD, :32]
    agg = ssum * dinv[:, None] + b2_ref[...]
    h2f = _bn_relu(agg[:N], gm2_ref[...], bt2_ref[...])
    cat = jnp.concatenate([x_ref[...], h1f_ref[...], h2f], axis=1)
    jf = jnp.maximum(
        jnp.dot(cat, wf_ref[...], preferred_element_type=jnp.float32)
        + bf_ref[...], 0.0)
    t = jnp.maximum(
        jnp.dot(jf, wj1_ref[...], preferred_element_type=jnp.float32)
        + bj1_ref[...], 0.0)
    out_ref[...] = (jnp.dot(t, wj2_ref[...], preferred_element_type=jnp.float32)
                    + bj2_ref[...])


def kernel(line_features, junction_features, line2junction_idx,
           junction_logits, line_logits,
           W1, b1, g1, be1, W2, b2, g2, be2, Wf, bf, Wj1, bj1, Wj2, bj2):
    f32 = jnp.float32
    i32 = jnp.int32

    pad_idx = (N + (jnp.arange(LPAD - L, dtype=i32) % 16)).astype(i32)
    A = jnp.concatenate([line2junction_idx[:, 0], pad_idx]).reshape(NW, NBL, BB)
    B = jnp.concatenate([line2junction_idx[:, 1], pad_idx]).reshape(NW, NBL, BB)
    keys = jnp.minimum(A, B) * NPAD + jnp.maximum(A, B)
    ids = jnp.arange(LPAD, dtype=i32).reshape(NW, NBL, BB)
    srce = jnp.concatenate([A, B], axis=1)
    dste = jnp.concatenate([B, A], axis=1)

    xpad = jnp.concatenate([junction_features, jnp.zeros((NPAD - N, 128), f32)])
    zeros1 = jnp.zeros((NPAD,), f32)
    zacc = jnp.zeros((ACCROWS, 128), f32)
    wj2p = jnp.concatenate([Wj2, jnp.zeros((32, 128 - 3), f32)], axis=1)
    bj2p = jnp.concatenate([bj2, jnp.zeros((128 - 3,), f32)])

    table = _sc_write_ids(keys, ids)
    src2, degp = _sc_readback_deg(keys, srce, dste, table, zeros1)

    g1pad = pl.pallas_call(
        _tc_pre,
        out_shape=jax.ShapeDtypeStruct((2 * NPAD, 128), f32),
    )(xpad, W1, degp)

    acc1p = _sc_agg1(g1pad, src2, dste, zacc)
    acc1 = (acc1p[:, 0] + acc1p[:, 1]).reshape(2, ACCROWS * 2, 64)
    acc1 = acc1[:, : NPAD // 2].reshape(NPAD, 64)

    h1f, g2pad = pl.pallas_call(
        _tc_mid,
        out_shape=(
            jax.ShapeDtypeStruct((N, 64), f32),
            jax.ShapeDtypeStruct((4 * NPAD, 128), f32),
        ),
    )(acc1, g1pad, degp, b1.reshape(1, 64), g1.reshape(1, 64),
      be1.reshape(1, 64), W2)

    acc2p = _sc_agg2(g2pad, src2, dste, zacc)
    acc2 = (acc2p[0, 0] + acc2p[0, 1]).reshape(ACCROWS * 4, 32)[:NPAD]

    out = pl.pallas_call(
        _tc_head,
        out_shape=jax.ShapeDtypeStruct((N, 128), f32),
    )(acc2, g2pad, degp, b2.reshape(1, 32), g2.reshape(1, 32),
      be2.reshape(1, 32), junction_features, h1f, Wf, bf.reshape(1, 128),
      Wj1, bj1.reshape(1, 32), wj2p, bj2p.reshape(1, 128))

    return (line_logits, out[:, :3])
